# Initial kernel scaffold; baseline (speedup 1.0000x reference)
import jax
import jax.numpy as jnp
from jax.experimental import pallas as pl


def kernel(anchor_ids, var_ids, edge_index, edge_type, batch_idx, targets, entity_table, mode_table, basis, comp, root, bias):
    raise NotImplementedError("write your pallas kernel here")


# speedup vs baseline: 11.2768x; 11.2768x over previous
import functools

import jax
import jax.numpy as jnp
from jax import lax
from jax.experimental import pallas as pl
from jax.experimental.pallas import tpu as pltpu
from jax.experimental.pallas import tpu_sc as plsc

D = 128
R = 20
NB = 10
B = 4096
NA = 2
NV = 4
NN = NA + NV
N = B * NN
E = 196608
QB = R + 1
WCOLS = QB * D

NC = 2
NS = 16
NW = NC * NS
NQ = 4
NQH = N // NQ
NQP = NQH + 128
EPT = E // NS
CH = 128
EPTC = EPT + 256
NCH2 = EPTC // CH
GPT = B // NW

_mesh = plsc.VectorSubcoreMesh(core_axis_name="c", subcore_axis_name="s")
_sc_params = pltpu.CompilerParams(needs_layout_passes=False)


def _wprep_body(comp_ref, basis_ref, root_ref, w_ref):
    for r in range(R):
        acc = comp_ref[r, 0] * basis_ref[0]
        for b in range(1, NB):
            acc = acc + comp_ref[r, b] * basis_ref[b]
        w_ref[:, r * D:(r + 1) * D] = acc
    w_ref[:, R * D:] = root_ref[...]


def _make_wbig(comp, basis, root):
    return pl.pallas_call(
        _wprep_body,
        in_specs=[
            pl.BlockSpec(memory_space=pltpu.SMEM),
            pl.BlockSpec(memory_space=pltpu.VMEM),
            pl.BlockSpec(memory_space=pltpu.VMEM),
        ],
        out_specs=pl.BlockSpec(memory_space=pltpu.VMEM),
        out_shape=jax.ShapeDtypeStruct((D, WCOLS), jnp.float32),
    )(comp, basis, root)


_BN = 256


def _mm_body(x_ref, w_ref, o_ref):
    o_ref[...] = jnp.dot(x_ref[...], w_ref[...],
                         preferred_element_type=jnp.float32)


def _matmul_xw(x, wbig):
    return pl.pallas_call(
        _mm_body,
        grid=(N // _BN,),
        in_specs=[
            pl.BlockSpec((_BN, D), lambda i: (i, 0)),
            pl.BlockSpec((D, WCOLS), lambda i: (0, 0)),
        ],
        out_specs=pl.BlockSpec((_BN, WCOLS), lambda i: (i, 0)),
        out_shape=jax.ShapeDtypeStruct((N, WCOLS), jnp.float32),
    )(x, wbig)


def _fused_body(agg_ref, cnt_ref, xprev_ref, bias_ref, w_ref, o_ref):
    rcp = 1.0 / jnp.maximum(cnt_ref[:, 0:1], 1.0)
    x1 = jax.nn.relu(agg_ref[...] * rcp + xprev_ref[...] + bias_ref[...])
    o_ref[...] = jnp.dot(x1, w_ref[...], preferred_element_type=jnp.float32)


def _fused_post_matmul(agg, cnt, xall_prev, bias2d, wbig):
    return pl.pallas_call(
        _fused_body,
        grid=(N // _BN,),
        in_specs=[
            pl.BlockSpec((_BN, D), lambda i: (i, 0)),
            pl.BlockSpec((_BN, D), lambda i: (i, 0)),
            pl.BlockSpec((_BN, D), lambda i: (i, R)),
            pl.BlockSpec((1, D), lambda i: (0, 0)),
            pl.BlockSpec((D, WCOLS), lambda i: (0, 0)),
        ],
        out_specs=pl.BlockSpec((_BN, WCOLS), lambda i: (i, 0)),
        out_shape=jax.ShapeDtypeStruct((N, WCOLS), jnp.float32),
    )(agg, cnt, xall_prev, bias2d, wbig)


def _post_body(agg_ref, cnt_ref, xprev_ref, bias_ref, o_ref):
    rcp = 1.0 / jnp.maximum(cnt_ref[:, 0:1], 1.0)
    o_ref[...] = agg_ref[...] * rcp + xprev_ref[...] + bias_ref[...]


def _final_post(agg, cnt, xall_prev, bias2d):
    return pl.pallas_call(
        _post_body,
        grid=(N // _BN,),
        in_specs=[
            pl.BlockSpec((_BN, D), lambda i: (i, 0)),
            pl.BlockSpec((_BN, D), lambda i: (i, 0)),
            pl.BlockSpec((_BN, D), lambda i: (i, R)),
            pl.BlockSpec((1, D), lambda i: (0, 0)),
        ],
        out_specs=pl.BlockSpec((_BN, D), lambda i: (i, 0)),
        out_shape=jax.ShapeDtypeStruct((N, D), jnp.float32),
    )(agg, cnt, xall_prev, bias2d)


_BG = 512


def _readout_body(xv_ref, p_ref, t_ref, o_ref):
    g = jnp.dot(xv_ref[...], p_ref[...], preferred_element_type=jnp.float32)
    t = t_ref[...]
    num = jnp.sum(g * t, axis=1, keepdims=True)
    den = jnp.sqrt(jnp.sum(g * g, axis=1, keepdims=True)) * \
        jnp.sqrt(jnp.sum(t * t, axis=1, keepdims=True))
    o_ref[...] = num / jnp.maximum(den, 1e-8)


def _readout(xv, pmat, trows):
    return pl.pallas_call(
        _readout_body,
        grid=(B // _BG,),
        in_specs=[
            pl.BlockSpec((_BG, NN * D), lambda i: (i, 0)),
            pl.BlockSpec((NN * D, D), lambda i: (0, 0)),
            pl.BlockSpec((_BG, D), lambda i: (i, 0)),
        ],
        out_specs=pl.BlockSpec((_BG, 1), lambda i: (i, 0)),
        out_shape=jax.ShapeDtypeStruct((B, 1), jnp.float32),
    )(xv, pmat, trows)


@functools.partial(
    pl.kernel,
    out_type=[
        jax.ShapeDtypeStruct((N, D), jnp.float32),
        jax.ShapeDtypeStruct((B, D), jnp.float32),
    ],
    mesh=_mesh,
    scratch_types=[
        pltpu.VMEM((CH,), jnp.int32),
        pltpu.VMEM((CH,), jnp.int32),
        pltpu.VMEM((CH, D), jnp.float32),
        pltpu.SemaphoreType.DMA,
    ],
    compiler_params=_sc_params,
)
def _build_x0(aflat_hbm, vflat_hbm, tgt_hbm, ent_hbm, mode_hbm,
              x0_hbm, t_hbm, id_buf, pos_buf, rows, sem):
    c = lax.axis_index("c")
    s = lax.axis_index("s")
    wid = s * NC + c
    gbase = wid * GPT
    lane = lax.iota(jnp.int32, 16)

    for ck in range(NA * GPT // CH):
        fbase = gbase * NA + ck * CH

        def abody(i, _):
            f = fbase + i * 16 + lane
            pos_buf[pl.ds(i * 16, 16)] = (f >> 1) * NN + (f & 1)
            return 0
        lax.fori_loop(0, CH // 16, abody, 0)
        pltpu.sync_copy(aflat_hbm.at[pl.ds(fbase, CH)], id_buf)
        pltpu.async_copy(ent_hbm.at[id_buf], rows, sem).wait()
        pltpu.sync_copy(rows, x0_hbm.at[pos_buf])

    for ck in range(NV * GPT // CH):
        fbase = gbase * NV + ck * CH

        def vbody(i, _):
            f = fbase + i * 16 + lane
            pos_buf[pl.ds(i * 16, 16)] = (f >> 2) * NN + NA + (f & 3)
            return 0
        lax.fori_loop(0, CH // 16, vbody, 0)
        pltpu.sync_copy(vflat_hbm.at[pl.ds(fbase, CH)], id_buf)
        pltpu.async_copy(mode_hbm.at[id_buf], rows, sem).wait()
        pltpu.sync_copy(rows, x0_hbm.at[pos_buf])

    pltpu.sync_copy(tgt_hbm.at[pl.ds(gbase, GPT)], id_buf)
    pltpu.async_copy(ent_hbm.at[id_buf], rows, sem).wait()
    pltpu.sync_copy(rows, t_hbm.at[pl.ds(gbase, GPT)])


@functools.partial(
    pl.kernel,
    out_type=[
        jax.ShapeDtypeStruct((NC, 2, NS, EPTC), jnp.int32),
        jax.ShapeDtypeStruct((NC, 2, NS, EPTC), jnp.int32),
        jax.ShapeDtypeStruct((NC, 2, NS, 16), jnp.int32),
    ],
    mesh=_mesh,
    scratch_types=[
        pltpu.VMEM((EPT,), jnp.int32),
        pltpu.VMEM((EPT,), jnp.int32),
        pltpu.VMEM((EPT,), jnp.int32),
        pltpu.VMEM((EPTC,), jnp.int32),
        pltpu.VMEM((EPTC,), jnp.int32),
        pltpu.VMEM((EPTC,), jnp.int32),
        pltpu.VMEM((EPTC,), jnp.int32),
        pltpu.VMEM((16,), jnp.int32),
    ],
    compiler_params=_sc_params,
)
def _compact(src_hbm, dst_hbm, et_hbm,
             rows_hbm, dstl_hbm, counts_hbm,
             src_all, dst_all, et_all, rows_l0, dst_l0, rows_l1, dst_l1,
             cbuf):
    c = lax.axis_index("c")
    s = lax.axis_index("s")
    ebase = s * EPT
    pltpu.sync_copy(src_hbm.at[pl.ds(ebase, EPT)], src_all)
    pltpu.sync_copy(dst_hbm.at[pl.ds(ebase, EPT)], dst_all)
    pltpu.sync_copy(et_hbm.at[pl.ds(ebase, EPT)], et_all)

    zero16 = jnp.zeros((16,), jnp.int32)
    lane = lax.iota(jnp.int32, 16)

    def prefill(i, _):
        sl = pl.ds(i * 16, 16)
        dump = NQH + ((i * 16 + lane) & 127)
        rows_l0[sl] = zero16
        dst_l0[sl] = dump
        rows_l1[sl] = zero16
        dst_l1[sl] = dump
        return 0
    lax.fori_loop(0, EPTC // 16, prefill, 0)

    lo0 = (2 * c) * NQH

    def compact_step(i, carry):
        pv0, pv1 = carry
        sl = pl.ds(i * 16, 16)
        dv = dst_all[sl] - lo0
        rowv = src_all[sl] * QB + et_all[sl]
        m0 = (dv >= 0) & (dv < NQH)
        dv1 = dv - NQH
        m1 = (dv1 >= 0) & (dv1 < NQH)
        offs0 = pv0 + plsc.cumsum(m0.astype(jnp.int32)) - 1
        plsc.store_scatter(rows_l0, [offs0], rowv, mask=m0)
        plsc.store_scatter(dst_l0, [offs0], dv, mask=m0)
        offs1 = pv1 + plsc.cumsum(m1.astype(jnp.int32)) - 1
        plsc.store_scatter(rows_l1, [offs1], rowv, mask=m1)
        plsc.store_scatter(dst_l1, [offs1], dv1, mask=m1)
        pv0 = pv0 + plsc.all_reduce_population_count(m0)
        pv1 = pv1 + plsc.all_reduce_population_count(m1)
        return pv0, pv1

    pv0, pv1 = lax.fori_loop(0, EPT // 16, compact_step, (zero16, zero16))

    cbuf[...] = ((pv0 + 255) >> 8) * 2
    pltpu.sync_copy(cbuf, counts_hbm.at[c, 0, s])
    cbuf[...] = ((pv1 + 255) >> 8) * 2
    pltpu.sync_copy(cbuf, counts_hbm.at[c, 1, s])
    pltpu.sync_copy(rows_l0, rows_hbm.at[c, 0, s])
    pltpu.sync_copy(dst_l0, dstl_hbm.at[c, 0, s])
    pltpu.sync_copy(rows_l1, rows_hbm.at[c, 1, s])
    pltpu.sync_copy(dst_l1, dstl_hbm.at[c, 1, s])


def _edge_pass(c, s, p, xrows_hbm, rows_hbm, dstl_hbm, counts_hbm, zacc_hbm,
               agg_hbm, acc, idx0, idx1, dq0, dq1, cbuf, rows0, rows1,
               sem0, sem1):
    pltpu.sync_copy(zacc_hbm, rows0)
    zb = s * (NQP // NS)
    for k in range(3):
        pltpu.sync_copy(rows0, acc.at[pl.ds(zb + k * CH, CH)])
    pltpu.sync_copy(rows0.at[pl.ds(0, 8)], acc.at[pl.ds(zb + 3 * CH, 8)])
    plsc.subcore_barrier()

    q = 2 * p + c
    lin = q * NS + s
    pltpu.sync_copy(counts_hbm.at[pl.ds(lin * 16, 16)], cbuf)
    nch = cbuf[...][0]
    lbase = lin * EPTC

    def chunk_pair(j2, _):
        j = j2 * 2
        pltpu.sync_copy(rows_hbm.at[pl.ds(lbase + j * CH, CH)], idx0)
        pltpu.sync_copy(rows_hbm.at[pl.ds(lbase + (j + 1) * CH, CH)], idx1)
        pltpu.sync_copy(dstl_hbm.at[pl.ds(lbase + j * CH, CH)], dq0)
        pltpu.sync_copy(dstl_hbm.at[pl.ds(lbase + (j + 1) * CH, CH)], dq1)
        cp0 = pltpu.async_copy(xrows_hbm.at[idx0], rows0, sem0)
        cp1 = pltpu.async_copy(xrows_hbm.at[idx1], rows1, sem1)
        cp0.wait()
        pltpu.sync_copy(rows0, acc.at[dq0], add=True)
        cp1.wait()
        pltpu.sync_copy(rows1, acc.at[dq1], add=True)
        return 0

    lax.fori_loop(0, nch >> 1, chunk_pair, 0)
    plsc.subcore_barrier()

    wb = s * (NQH // NS)
    obase = c * NQH + wb
    for k in range(3):
        pltpu.sync_copy(acc.at[pl.ds(wb + k * CH, CH)], rows0)
        pltpu.sync_copy(rows0, agg_hbm.at[pl.ds(obase + k * CH, CH)])


_edge_scratch = [
    pltpu.VMEM((CH,), jnp.int32),
    pltpu.VMEM((CH,), jnp.int32),
    pltpu.VMEM((CH,), jnp.int32),
    pltpu.VMEM((CH,), jnp.int32),
    pltpu.VMEM((16,), jnp.int32),
    pltpu.VMEM((CH, D), jnp.float32),
    pltpu.VMEM((CH, D), jnp.float32),
    pltpu.SemaphoreType.DMA,
    pltpu.SemaphoreType.DMA,
    pltpu.VMEM_SHARED((NQP, D), jnp.float32),
]


def _make_edge(p):
    @functools.partial(
        pl.kernel,
        out_type=[jax.ShapeDtypeStruct((NC * NQH, D), jnp.float32)],
        mesh=_mesh,
        scratch_types=_edge_scratch,
        compiler_params=_sc_params,
    )
    def _edge(xrows_hbm, rows_hbm, dstl_hbm, counts_hbm, zacc_hbm,
              agg_hbm,
              idx0, idx1, dq0, dq1, cbuf, rows0, rows1, sem0, sem1,
              acc):
        c = lax.axis_index("c")
        s = lax.axis_index("s")
        _edge_pass(c, s, p, xrows_hbm, rows_hbm, dstl_hbm, counts_hbm,
                   zacc_hbm, agg_hbm, acc, idx0, idx1, dq0, dq1, cbuf,
                   rows0, rows1, sem0, sem1)
    return _edge


_edge_p0 = _make_edge(0)
_edge_p1 = _make_edge(1)


def _make_cnt(p):
    @functools.partial(
        pl.kernel,
        out_type=[jax.ShapeDtypeStruct((NC * NQH, D), jnp.float32)],
        mesh=_mesh,
        scratch_types=[
            pltpu.VMEM((CH,), jnp.int32),
            pltpu.VMEM((CH,), jnp.int32),
            pltpu.VMEM((16,), jnp.int32),
            pltpu.VMEM((CH, D), jnp.float32),
            pltpu.VMEM((CH, D), jnp.float32),
            pltpu.VMEM_SHARED((NQP, D), jnp.float32),
        ],
        compiler_params=_sc_params,
    )
    def _cntk(dstl_hbm, counts_hbm, zacc_hbm, ones_hbm,
              cnt_hbm,
              dq0, dq1, cbuf, ones128, stage, acc):
        c = lax.axis_index("c")
        s = lax.axis_index("s")
        pltpu.sync_copy(zacc_hbm, stage)
        zb = s * (NQP // NS)
        for k in range(3):
            pltpu.sync_copy(stage, acc.at[pl.ds(zb + k * CH, CH)])
        pltpu.sync_copy(stage.at[pl.ds(0, 8)], acc.at[pl.ds(zb + 3 * CH, 8)])
        pltpu.sync_copy(ones_hbm, ones128)
        plsc.subcore_barrier()

        q = 2 * p + c
        lin = q * NS + s
        pltpu.sync_copy(counts_hbm.at[pl.ds(lin * 16, 16)], cbuf)
        nch = cbuf[...][0]
        lbase = lin * EPTC

        def chunk_pair(j2, _):
            j = j2 * 2
            pltpu.sync_copy(dstl_hbm.at[pl.ds(lbase + j * CH, CH)], dq0)
            pltpu.sync_copy(dstl_hbm.at[pl.ds(lbase + (j + 1) * CH, CH)], dq1)
            pltpu.sync_copy(ones128, acc.at[dq0], add=True)
            pltpu.sync_copy(ones128, acc.at[dq1], add=True)
            return 0

        lax.fori_loop(0, nch >> 1, chunk_pair, 0)
        plsc.subcore_barrier()

        wb = s * (NQH // NS)
        obase = c * NQH + wb
        for k in range(3):
            pltpu.sync_copy(acc.at[pl.ds(wb + k * CH, CH)], stage)
            pltpu.sync_copy(stage, cnt_hbm.at[pl.ds(obase + k * CH, CH)])
    return _cntk


_cnt_p0 = _make_cnt(0)
_cnt_p1 = _make_cnt(1)


def kernel(anchor_ids, var_ids, edge_index, edge_type, batch_idx, targets,
           entity_table, mode_table, basis, comp, root, bias):
    del batch_idx
    aflat = anchor_ids.reshape(-1).astype(jnp.int32)
    vflat = var_ids.reshape(-1).astype(jnp.int32)
    tgt = targets.astype(jnp.int32)
    src = edge_index[0].astype(jnp.int32)
    dst = edge_index[1].astype(jnp.int32)
    et = edge_type.astype(jnp.int32)
    bias2d = bias.reshape(1, D)

    zacc = jnp.zeros((CH, D), jnp.float32)
    ones = jnp.ones((CH, D), jnp.float32)
    pmat = jnp.tile(jnp.eye(D, dtype=jnp.float32), (NN, 1))

    wbig = _make_wbig(comp, basis, root)
    x0, trows = _build_x0(aflat, vflat, tgt, entity_table, mode_table)
    rows_l, dst_l, counts = _compact(src, dst, et)
    rows_l = rows_l.reshape(NC * 2 * NS * EPTC)
    dst_l = dst_l.reshape(NC * 2 * NS * EPTC)
    counts = counts.reshape(NC * 2 * NS * 16)

    (cnta,) = _cnt_p0(dst_l, counts, zacc, ones)
    (cntb,) = _cnt_p1(dst_l, counts, zacc, ones)
    cnt = jnp.concatenate([cnta, cntb], axis=0)

    xall1 = _matmul_xw(x0, wbig)
    xr1 = xall1.reshape(N * QB, D)
    (agg1a,) = _edge_p0(xr1, rows_l, dst_l, counts, zacc)
    (agg1b,) = _edge_p1(xr1, rows_l, dst_l, counts, zacc)
    agg1 = jnp.concatenate([agg1a, agg1b], axis=0)
    xall2 = _fused_post_matmul(agg1, cnt, xall1, bias2d, wbig)
    xr2 = xall2.reshape(N * QB, D)
    (agg2a,) = _edge_p0(xr2, rows_l, dst_l, counts, zacc)
    (agg2b,) = _edge_p1(xr2, rows_l, dst_l, counts, zacc)
    agg2 = jnp.concatenate([agg2a, agg2b], axis=0)
    x2 = _final_post(agg2, cnt, xall2, bias2d)
    out = _readout(x2.reshape(B, NN * D), pmat, trows)
    return out.reshape(B)
